# baseline (speedup 1.0000x reference)
import jax
import jax.numpy as jnp
from jax.experimental import pallas as pl

B, T, L = 8, 32, 64
D = 768
H = 12
dh = D // H
TT = 8
NT = T // TT


def _body(spk_row_ref, spk_col_ref, tok_ref, edu_ref, wqt_ref, wkt_ref,
          wvt_ref, out_ref):
    f32 = jnp.float32
    bf16 = jnp.bfloat16
    i = pl.program_id(1)
    t0 = i * TT

    tok = tok_ref[0]
    q = jax.lax.dot(tok.astype(bf16), wqt_ref[:].astype(bf16),
                    preferred_element_type=f32).astype(bf16)
    edu = edu_ref[0]
    k = jax.lax.dot(edu.astype(bf16), wkt_ref[:].astype(bf16),
                    preferred_element_type=f32).astype(bf16)
    v = jax.lax.dot(edu.astype(bf16), wvt_ref[:].astype(bf16),
                    preferred_element_type=f32).astype(bf16)

    spk_row = spk_row_ref[0]
    spk_col = spk_col_ref[0]
    same = spk_col == spk_row
    trow = jax.lax.broadcasted_iota(jnp.int32, (TT, T), 0) + t0
    ucol = jax.lax.broadcasted_iota(jnp.int32, (TT, T), 1)
    mask = (same & (ucol <= trow)).astype(f32)
    mask = mask.reshape(TT, 1, T)

    parts = []
    for h in range(H):
        sl = slice(h * dh, (h + 1) * dh)
        q_h = q[:, sl]
        k_h = k[:, sl]
        v_h = v[:, sl]
        s = jax.lax.dot_general(q_h, k_h, (((1,), (1,)), ((), ())),
                                preferred_element_type=f32)
        s = (s.reshape(TT, L, T) * mask).astype(bf16)
        a_h = jax.lax.dot(s.reshape(TT * L, T), v_h,
                          preferred_element_type=f32)
        parts.append(a_h)
    out_ref[0] = tok + jnp.concatenate(parts, axis=1)


def kernel(input_ids, speaker_names, token_embeddings, edu_embeddings,
           Wk, Wv, Wq):
    tok = token_embeddings.reshape(B, T * L, D)
    spk = speaker_names.astype(jnp.int32)
    spk_row = spk.reshape(B, 1, T)
    spk_col = spk.reshape(B, T, 1)

    out = pl.pallas_call(
        _body,
        grid=(B, NT),
        in_specs=[
            pl.BlockSpec((1, 1, T), lambda b, i: (b, 0, 0)),
            pl.BlockSpec((1, TT, 1), lambda b, i: (b, i, 0)),
            pl.BlockSpec((1, TT * L, D), lambda b, i: (b, i, 0)),
            pl.BlockSpec((1, T, D), lambda b, i: (b, 0, 0)),
            pl.BlockSpec((D, D), lambda b, i: (0, 0)),
            pl.BlockSpec((D, D), lambda b, i: (0, 0)),
            pl.BlockSpec((D, D), lambda b, i: (0, 0)),
        ],
        out_specs=pl.BlockSpec((1, TT * L, D), lambda b, i: (b, i, 0)),
        out_shape=jax.ShapeDtypeStruct((B, T * L, D), jnp.float32),
    )(spk_row, spk_col, tok, edu_embeddings, Wq.T, Wk.T, Wv.T)
    return out.reshape(B, T, L, D)


# speedup vs baseline: 1.1103x; 1.1103x over previous
import jax
import jax.numpy as jnp
from jax.experimental import pallas as pl

B, T, L = 8, 32, 64
D = 768
H = 12
dh = D // H
TT = 16
NT = T // TT


def _body(spk_row_ref, spk_col_ref, tok_ref, edu_ref, wqt_ref, wkt_ref,
          wvt_ref, out_ref):
    f32 = jnp.float32
    bf16 = jnp.bfloat16
    i = pl.program_id(1)
    t0 = i * TT

    tok = tok_ref[0]
    q = jax.lax.dot(tok.astype(bf16), wqt_ref[:].astype(bf16),
                    preferred_element_type=f32).astype(bf16)
    edu = edu_ref[0]
    k = jax.lax.dot(edu.astype(bf16), wkt_ref[:].astype(bf16),
                    preferred_element_type=f32).astype(bf16)
    v = jax.lax.dot(edu.astype(bf16), wvt_ref[:].astype(bf16),
                    preferred_element_type=f32).astype(bf16)

    spk_row = spk_row_ref[0]
    spk_col = spk_col_ref[0]
    same = spk_col == spk_row
    trow = jax.lax.broadcasted_iota(jnp.int32, (TT, T), 0) + t0
    ucol = jax.lax.broadcasted_iota(jnp.int32, (TT, T), 1)
    mask = (same & (ucol <= trow)).astype(f32)
    mask = mask.reshape(TT, 1, T)

    parts = []
    for h in range(H):
        sl = slice(h * dh, (h + 1) * dh)
        q_h = q[:, sl]
        k_h = k[:, sl]
        v_h = v[:, sl]
        s = jax.lax.dot_general(q_h, k_h, (((1,), (1,)), ((), ())),
                                preferred_element_type=f32)
        s = (s.reshape(TT, L, T) * mask).astype(bf16)
        a_h = jax.lax.dot(s.reshape(TT * L, T), v_h,
                          preferred_element_type=f32)
        parts.append(a_h)
    out_ref[0] = tok + jnp.concatenate(parts, axis=1)


def kernel(input_ids, speaker_names, token_embeddings, edu_embeddings,
           Wk, Wv, Wq):
    tok = token_embeddings.reshape(B, T * L, D)
    spk = speaker_names.astype(jnp.int32)
    spk_row = spk.reshape(B, 1, T)
    spk_col = spk.reshape(B, T, 1)

    out = pl.pallas_call(
        _body,
        grid=(B, NT),
        in_specs=[
            pl.BlockSpec((1, 1, T), lambda b, i: (b, 0, 0)),
            pl.BlockSpec((1, TT, 1), lambda b, i: (b, i, 0)),
            pl.BlockSpec((1, TT * L, D), lambda b, i: (b, i, 0)),
            pl.BlockSpec((1, T, D), lambda b, i: (b, 0, 0)),
            pl.BlockSpec((D, D), lambda b, i: (0, 0)),
            pl.BlockSpec((D, D), lambda b, i: (0, 0)),
            pl.BlockSpec((D, D), lambda b, i: (0, 0)),
        ],
        out_specs=pl.BlockSpec((1, TT * L, D), lambda b, i: (b, i, 0)),
        out_shape=jax.ShapeDtypeStruct((B, T * L, D), jnp.float32),
    )(spk_row, spk_col, tok, edu_embeddings, Wq.T, Wk.T, Wv.T)
    return out.reshape(B, T, L, D)
